# baseline (speedup 1.0000x reference)
import functools

import jax
import jax.numpy as jnp
from jax.experimental import pallas as pl
from jax.experimental.pallas import tpu as pltpu

_VMEM_LIMIT = 56 * 1024 * 1024


def _lstm_steps(ig_at, h, c, whh, t_blk, bb, H):
    def sig(v):
        return 0.5 * jnp.tanh(0.5 * v) + 0.5

    hs = []
    for i in range(t_blk):
        g = ig_at(i) + jnp.dot(h.astype(jnp.bfloat16), whh,
                               preferred_element_type=jnp.float32)
        s_if = sig(g[:, 0 * H:2 * H])
        g_g = jnp.tanh(g[:, 2 * H:3 * H])
        o_g = sig(g[:, 3 * H:4 * H])
        c = s_if[:, H:2 * H] * c + s_if[:, 0 * H:H] * g_g
        h = o_g * jnp.tanh(c)
        hs.append(h)
    return hs, h, c


def _pipelined_lstm_kernel(x_ref, wih0_ref, whh0_ref, b0_ref, wih1_ref,
                           whh1_ref, b1_ref, wfc_ref, bfc_ref,
                           fc0_ref, fc1_ref,
                           h0_scr, c0_scr, h1_scr, c1_scr, ig1_scr, hs0_scr,
                           *, t_blk, bb, hidden, n_blk):
    t = pl.program_id(0)
    H = hidden
    rows = t_blk * bb
    zero = jnp.zeros((), jnp.float32)

    h0_entry = jnp.where(t == 0, zero, h0_scr[...])
    h0 = h0_entry
    c0 = jnp.where(t == 0, zero, c0_scr[...])
    h1 = h1_scr[...]
    c1 = c1_scr[...]

    ig0 = jnp.dot(x_ref[...], wih0_ref[...],
                  preferred_element_type=jnp.float32) + b0_ref[...]
    whh0 = whh0_ref[...]
    whh1 = whh1_ref[...]

    def sig(v):
        return 0.5 * jnp.tanh(0.5 * v) + 0.5

    def gates(ig_i, h, whh):
        return ig_i + jnp.dot(h.astype(jnp.bfloat16), whh,
                              preferred_element_type=jnp.float32)

    def elem(g, c):
        s_if = sig(g[:, 0 * H:2 * H])
        g_g = jnp.tanh(g[:, 2 * H:3 * H])
        o_g = sig(g[:, 3 * H:4 * H])
        c = s_if[:, H:2 * H] * c + s_if[:, 0 * H:H] * g_g
        h = o_g * jnp.tanh(c)
        return h, c

    for i in range(t_blk):
        g0 = gates(ig0[i * bb:(i + 1) * bb, :], h0, whh0)
        g1 = gates(ig1_scr[i * bb:(i + 1) * bb, :], h1, whh1)
        h0, c0 = elem(g0, c0)
        hs0_scr[i * bb:(i + 1) * bb, :] = h0.astype(jnp.bfloat16)
        h1, c1 = elem(g1, c1)

    ig1_scr[...] = (jnp.dot(hs0_scr[...], wih1_ref[...],
                            preferred_element_type=jnp.float32)
                    + b1_ref[...]).astype(jnp.bfloat16)

    h0_scr[...] = h0
    c0_scr[...] = c0
    h1_scr[...] = jnp.where(t == 0, zero, h1)
    c1_scr[...] = jnp.where(t == 0, zero, c1)

    @pl.when(t == n_blk)
    def _head():
        wfc = wfc_ref[...]
        bfc = bfc_ref[...]
        fc1_ref[...] = jnp.dot(h1.astype(jnp.bfloat16), wfc,
                               preferred_element_type=jnp.float32) + bfc
        fc0_ref[...] = jnp.dot(h0_entry.astype(jnp.bfloat16), wfc,
                               preferred_element_type=jnp.float32) + bfc


@jax.jit
def kernel(x, w_ih_0, w_hh_0, b_ih_0, b_hh_0,
           w_ih_1, w_hh_1, b_ih_1, b_hh_1, w_fc, b_fc):
    B, S, D = x.shape
    H = w_hh_0.shape[1]
    C = w_fc.shape[0]
    G = 4 * H

    T_BLK = 16
    N_BLK = S // T_BLK

    bf16 = jnp.bfloat16
    x_tm = jnp.transpose(x, (1, 0, 2)).reshape(S * B, D).astype(bf16)
    wih0 = jnp.transpose(w_ih_0).astype(bf16)
    whh0 = jnp.transpose(w_hh_0).astype(bf16)
    b0 = (b_ih_0 + b_hh_0).reshape(1, G)
    wih1 = jnp.transpose(w_ih_1).astype(bf16)
    whh1 = jnp.transpose(w_hh_1).astype(bf16)
    b1 = (b_ih_1 + b_hh_1).reshape(1, G)
    wfc = jnp.transpose(w_fc).astype(bf16)
    bfc = b_fc.reshape(1, C)

    body = functools.partial(_pipelined_lstm_kernel, t_blk=T_BLK, bb=B,
                             hidden=H, n_blk=N_BLK)
    last = N_BLK - 1
    fc0, fc1 = pl.pallas_call(
        body,
        out_shape=(
            jax.ShapeDtypeStruct((B, C), jnp.float32),
            jax.ShapeDtypeStruct((B, C), jnp.float32),
        ),
        grid=(N_BLK + 1,),
        in_specs=[
            pl.BlockSpec((T_BLK * B, D), lambda t: (jnp.minimum(t, last), 0)),
            pl.BlockSpec((D, G), lambda t: (0, 0)),
            pl.BlockSpec((H, G), lambda t: (0, 0)),
            pl.BlockSpec((1, G), lambda t: (0, 0)),
            pl.BlockSpec((H, G), lambda t: (0, 0)),
            pl.BlockSpec((H, G), lambda t: (0, 0)),
            pl.BlockSpec((1, G), lambda t: (0, 0)),
            pl.BlockSpec((H, C), lambda t: (0, 0)),
            pl.BlockSpec((1, C), lambda t: (0, 0)),
        ],
        out_specs=[
            pl.BlockSpec((B, C), lambda t: (0, 0)),
            pl.BlockSpec((B, C), lambda t: (0, 0)),
        ],
        scratch_shapes=[
            pltpu.VMEM((B, H), jnp.float32),
            pltpu.VMEM((B, H), jnp.float32),
            pltpu.VMEM((B, H), jnp.float32),
            pltpu.VMEM((B, H), jnp.float32),
            pltpu.VMEM((T_BLK * B, G), jnp.bfloat16),
            pltpu.VMEM((T_BLK * B, H), jnp.bfloat16),
        ],
        compiler_params=pltpu.CompilerParams(
            dimension_semantics=("arbitrary",),
            vmem_limit_bytes=_VMEM_LIMIT,
        ),
    )(x_tm, wih0, whh0, b0, wih1, whh1, b1, wfc, bfc)

    return jnp.concatenate([fc0, fc1], axis=0)


# speedup vs baseline: 1.0049x; 1.0049x over previous
import functools

import jax
import jax.numpy as jnp
from jax.experimental import pallas as pl
from jax.experimental.pallas import tpu as pltpu

_VMEM_LIMIT = 56 * 1024 * 1024


def _pipelined_lstm_kernel(x_ref, wih0_ref, whh0_ref, b0_ref, wih1_ref,
                           whh1_ref, b1_ref, wfc_ref, bfc_ref,
                           fc0_ref, fc1_ref,
                           h0_scr, c0_scr, h1_scr, c1_scr, ig1_scr,
                           *, t_blk, bb, hidden, n_blk):
    t = pl.program_id(0)
    H = hidden
    zero = jnp.zeros((), jnp.float32)

    h0_entry = jnp.where(t == 0, zero, h0_scr[...])
    h0 = h0_entry
    c0 = jnp.where(t == 0, zero, c0_scr[...])
    h1 = h1_scr[...]
    c1 = c1_scr[...]

    ig0 = jnp.dot(x_ref[...], wih0_ref[...],
                  preferred_element_type=jnp.float32) + b0_ref[...]
    whh0 = whh0_ref[...]
    whh1 = whh1_ref[...]

    def sig(v):
        return 0.5 * jnp.tanh(0.5 * v) + 0.5

    def gates(ig_i, h, whh):
        return ig_i + jnp.dot(h.astype(jnp.bfloat16), whh,
                              preferred_element_type=jnp.float32)

    def elem(g, c):
        s_if = sig(g[:, 0 * H:2 * H])
        g_g = jnp.tanh(g[:, 2 * H:3 * H])
        o_g = sig(g[:, 3 * H:4 * H])
        c = s_if[:, H:2 * H] * c + s_if[:, 0 * H:H] * g_g
        h = o_g * jnp.tanh(c)
        return h, c

    hs0 = []
    for i in range(t_blk):
        g0 = gates(ig0[i * bb:(i + 1) * bb, :], h0, whh0)
        g1 = gates(ig1_scr[i * bb:(i + 1) * bb, :], h1, whh1)
        h0, c0 = elem(g0, c0)
        hs0.append(h0)
        h1, c1 = elem(g1, c1)

    x1 = jnp.concatenate(hs0, axis=0).astype(jnp.bfloat16)
    ig1_scr[...] = (jnp.dot(x1, wih1_ref[...],
                            preferred_element_type=jnp.float32)
                    + b1_ref[...]).astype(jnp.bfloat16)

    h0_scr[...] = h0
    c0_scr[...] = c0
    h1_scr[...] = jnp.where(t == 0, zero, h1)
    c1_scr[...] = jnp.where(t == 0, zero, c1)

    @pl.when(t == n_blk)
    def _head():
        wfc = wfc_ref[...]
        bfc = bfc_ref[...]
        fc1_ref[...] = jnp.dot(h1.astype(jnp.bfloat16), wfc,
                               preferred_element_type=jnp.float32) + bfc
        fc0_ref[...] = jnp.dot(h0_entry.astype(jnp.bfloat16), wfc,
                               preferred_element_type=jnp.float32) + bfc


@jax.jit
def kernel(x, w_ih_0, w_hh_0, b_ih_0, b_hh_0,
           w_ih_1, w_hh_1, b_ih_1, b_hh_1, w_fc, b_fc):
    B, S, D = x.shape
    H = w_hh_0.shape[1]
    C = w_fc.shape[0]
    G = 4 * H

    T_BLK = 16
    N_BLK = S // T_BLK

    bf16 = jnp.bfloat16
    x_tm = jnp.transpose(x, (1, 0, 2)).reshape(S * B, D).astype(bf16)
    wih0 = jnp.transpose(w_ih_0).astype(bf16)
    whh0 = jnp.transpose(w_hh_0).astype(bf16)
    b0 = (b_ih_0 + b_hh_0).reshape(1, G)
    wih1 = jnp.transpose(w_ih_1).astype(bf16)
    whh1 = jnp.transpose(w_hh_1).astype(bf16)
    b1 = (b_ih_1 + b_hh_1).reshape(1, G)
    wfc = jnp.transpose(w_fc).astype(bf16)
    bfc = b_fc.reshape(1, C)

    body = functools.partial(_pipelined_lstm_kernel, t_blk=T_BLK, bb=B,
                             hidden=H, n_blk=N_BLK)
    last = N_BLK - 1
    fc0, fc1 = pl.pallas_call(
        body,
        out_shape=(
            jax.ShapeDtypeStruct((B, C), jnp.float32),
            jax.ShapeDtypeStruct((B, C), jnp.float32),
        ),
        grid=(N_BLK + 1,),
        in_specs=[
            pl.BlockSpec((T_BLK * B, D), lambda t: (jnp.minimum(t, last), 0)),
            pl.BlockSpec((D, G), lambda t: (0, 0)),
            pl.BlockSpec((H, G), lambda t: (0, 0)),
            pl.BlockSpec((1, G), lambda t: (0, 0)),
            pl.BlockSpec((H, G), lambda t: (0, 0)),
            pl.BlockSpec((H, G), lambda t: (0, 0)),
            pl.BlockSpec((1, G), lambda t: (0, 0)),
            pl.BlockSpec((H, C), lambda t: (0, 0)),
            pl.BlockSpec((1, C), lambda t: (0, 0)),
        ],
        out_specs=[
            pl.BlockSpec((B, C), lambda t: (0, 0)),
            pl.BlockSpec((B, C), lambda t: (0, 0)),
        ],
        scratch_shapes=[
            pltpu.VMEM((B, H), jnp.float32),
            pltpu.VMEM((B, H), jnp.float32),
            pltpu.VMEM((B, H), jnp.float32),
            pltpu.VMEM((B, H), jnp.float32),
            pltpu.VMEM((T_BLK * B, G), jnp.bfloat16),
        ],
        compiler_params=pltpu.CompilerParams(
            dimension_semantics=("arbitrary",),
            vmem_limit_bytes=_VMEM_LIMIT,
        ),
    )(x_tm, wih0, whh0, b0, wih1, whh1, b1, wfc, bfc)

    return jnp.concatenate([fc0, fc1], axis=0)
